# baseline (speedup 1.0000x reference)
import functools

import numpy as np
import jax
import jax.numpy as jnp
from jax import lax
from jax.experimental import pallas as pl
from jax.experimental.pallas import tpu as pltpu
from jax.experimental.pallas import tpu_sc as plsc

_B = 128
_C = 5
_S = 256
_H = 128
_L = 16
_NC, _NS = 2, 16
_NW = _NC * _NS
_RCH = 32
_CPT = _H // _RCH
_CPW = (_B * 2 * _CPT) // _NW
_HSLOTS = 8 * _L


def _sc_hist_body(cpw, seg_hbm, out_hbm, buf, hist, sem0, sem1):
    wid = lax.axis_index("s") * _NC + lax.axis_index("c")
    g0 = wid * cpw
    one = jnp.ones((_L,), jnp.float32)
    zf = jnp.zeros((_L,), jnp.float32)
    sems = (sem0, sem1)

    def chunk_addr(gg):
        b = gg >> 3
        tb = (gg >> 2) & 1
        r0 = tb * _H + (gg & 3) * _RCH
        return b, tb, r0

    def start(gg, slot):
        b, _, r0 = chunk_addr(gg)
        pltpu.async_copy(
            seg_hbm.at[b, :, pl.ds(r0, _RCH), :], buf.at[slot], sems[slot])

    def wait(gg, slot):
        b, _, r0 = chunk_addr(gg)
        pltpu.make_async_copy(
            seg_hbm.at[b, :, pl.ds(r0, _RCH), :], buf.at[slot],
            sems[slot]).wait()

    def compute(slot, accs):
        def sel4(r, col):
            xs = [buf[slot, c, r, pl.ds(col, _L)] for c in range(_C)]
            m = jnp.maximum(
                jnp.maximum(jnp.maximum(xs[0], xs[1]),
                            jnp.maximum(xs[2], xs[3])), xs[4])
            return [jnp.where(xs[cp + 1] == m, one, zf) for cp in range(4)]

        def half(lr):
            def row(r, a4):
                a4 = list(a4)
                for jp in range(4):
                    s_even = sel4(r, lr * _H + (2 * jp) * _L)
                    s_odd = sel4(r, lr * _H + (2 * jp + 1) * _L)
                    for cp in range(4):
                        a4[cp] = a4[cp] + (s_even[cp] + s_odd[cp])
                return tuple(a4)
            return row

        accl = lax.fori_loop(0, _RCH, half(0), tuple(accs[0:4]))
        accr = lax.fori_loop(0, _RCH, half(1), tuple(accs[4:8]))
        return accl + accr

    start(g0, 0)
    start(g0 + 1, 1)

    def outer(gp, accs):
        for slot in range(2):
            g = gp * 2 + slot
            gg = g0 + g
            wait(gg, slot)
            accs = compute(slot, accs)

            @pl.when(g + 2 < cpw)
            def _():
                start(gg + 2, slot)

            b, tb, _ = chunk_addr(gg)
            flush = (gg & 3) == 3

            @pl.when(flush)
            def _():
                for k in range(8):
                    hist[pl.ds(k * _L, _L)] = accs[k]
                pltpu.sync_copy(hist, out_hbm.at[b, pl.ds(tb * _HSLOTS, _HSLOTS)])

            accs = tuple(jnp.where(flush, zf, a) for a in accs)
        return accs

    lax.fori_loop(0, cpw // 2, outer, tuple(zf for _ in range(8)))


@functools.cache
def _get_sc_hist(bs):
    cpw = (bs * 2 * _CPT) // _NW
    return pl.kernel(
        functools.partial(_sc_hist_body, cpw),
        out_type=jax.ShapeDtypeStruct((bs, 2 * _HSLOTS), jnp.float32),
        mesh=plsc.VectorSubcoreMesh(core_axis_name="c", subcore_axis_name="s"),
        scratch_types=[
            pltpu.VMEM((2, _C, _RCH, _S), jnp.float32),
            pltpu.VMEM((_HSLOTS,), jnp.float32),
            pltpu.SemaphoreType.DMA,
            pltpu.SemaphoreType.DMA,
        ],
        compiler_params=pltpu.CompilerParams(needs_layout_passes=False),
    )


def _build_tc_selectors():
    st = np.zeros((_S, 2), np.float32)
    st[:_H, 0] = 1.0
    st[_H:, 1] = 1.0
    u8 = np.zeros((8, 4 * _S), np.float32)
    for tb in range(2):
        for cp in range(4):
            for r in range(tb * _H, (tb + 1) * _H):
                u8[cp * 2 + tb, cp * _S + r] = 1.0
    return st, u8


_ST, _U8 = _build_tc_selectors()


def _tc_hist_body(seg_ref, st_ref, u8_ref, o_ref):
    x = seg_ref[0]
    m = jnp.max(x, axis=0)
    eq = (x[1:] == m[None]).astype(jnp.float32)
    e = eq.reshape(4 * _S, _S)
    a = jnp.dot(e, st_ref[...], preferred_element_type=jnp.float32)
    o_ref[0] = jnp.dot(u8_ref[...], a, preferred_element_type=jnp.float32)


@functools.cache
def _get_tc_hist(bt):
    b0 = _B - bt
    return pl.pallas_call(
        _tc_hist_body,
        grid=(bt,),
        in_specs=[
            pl.BlockSpec((1, _C, _S, _S), lambda i: (i + b0, 0, 0, 0)),
            pl.BlockSpec((_S, 2), lambda i: (0, 0)),
            pl.BlockSpec((8, 4 * _S), lambda i: (0, 0)),
        ],
        out_specs=pl.BlockSpec((1, 8, 2), lambda i: (i, 0, 0)),
        out_shape=jax.ShapeDtypeStruct((bt, 8, 2), jnp.float32),
    )


def _build_selectors():
    g5 = np.zeros((2 * _HSLOTS, _C), np.float32)
    g20 = np.zeros((2 * _HSLOTS, 4 * _C), np.float32)
    g5t = np.zeros((16, _C), np.float32)
    g20t = np.zeros((16, 4 * _C), np.float32)
    b5 = np.zeros((1, _C), np.float32)
    b20 = np.zeros((1, 4 * _C), np.float32)
    b5[0, 0] = 1.0
    for tb in range(2):
        for lr in range(2):
            q = lr * 2 + tb
            b20[0, q * _C] = 1.0
            for cp in range(_C - 1):
                k = cp * 4 + tb * 2 + lr
                g5t[k, cp + 1] = 1.0 / float(_S * _S)
                g5t[k, 0] = -1.0 / float(_S * _S)
                g20t[k, q * _C + cp + 1] = 1.0 / float(_H * _H)
                g20t[k, q * _C] = -1.0 / float(_H * _H)
                for ln in range(_L):
                    j = tb * _HSLOTS + (lr * 4 + cp) * _L + ln
                    g5[j, cp + 1] = 1.0 / float(_S * _S)
                    g5[j, 0] = -1.0 / float(_S * _S)
                    g20[j, q * _C + cp + 1] = 1.0 / float(_H * _H)
                    g20[j, q * _C] = -1.0 / float(_H * _H)
    return g5, g20, g5t, g20t, b5, b20


_G5, _G20, _G5T, _G20T, _B5, _B20 = _build_selectors()


def _dot(a, b):
    return jnp.dot(a, b, preferred_element_type=jnp.float32)


def _count_dot(y, g):
    hi = jnp.floor(y * (1.0 / 128.0))
    lo = y - hi * 128.0
    return _dot(hi, g) * 128.0 + _dot(lo, g)


def _stage2_body(ysc_ref, ytc_ref, g5_ref, g20_ref, g5t_ref, g20t_ref,
                 b5_ref, b20_ref, m5_ref, iv5_ref, m20_ref, iv20_ref,
                 s_ref, o_ref):
    ysc = ysc_ref[...]
    ytc = ytc_ref[...]
    u5 = jnp.concatenate(
        [_count_dot(ysc, g5_ref[...]), _count_dot(ytc, g5t_ref[...])], axis=0)
    u20 = jnp.concatenate(
        [_count_dot(ysc, g20_ref[...]), _count_dot(ytc, g20t_ref[...])], axis=0)
    d5 = u5 + (b5_ref[...] - m5_ref[...])
    d20 = u20 + (b20_ref[...] - m20_ref[...])
    q5 = jnp.sum(d5 * _dot(d5, iv5_ref[...]), axis=1, keepdims=True)
    q20 = jnp.sum(d20 * _dot(d20, iv20_ref[...]), axis=1, keepdims=True)
    hvm = s_ref[:, 0:1]
    hvs = s_ref[:, 1:2]
    phvm = s_ref[:, 2:3]
    phvs = s_ref[:, 3:4]
    o_ref[...] = (jnp.sqrt(q5) - hvm) / hvs + (jnp.sqrt(q20) - phvm) / phvs


@functools.cache
def _get_stage2():
    return pl.pallas_call(
        _stage2_body,
        out_shape=jax.ShapeDtypeStruct((_B, 1), jnp.float32),
    )


_BS = 8


def kernel(segmap, hist_mean, hist_invcov, patch_hist_mean, patch_hist_invcov,
           hist_val_mean, hist_val_std, patch_hist_val_mean, patch_hist_val_std):
    raw_sc = _get_sc_hist(_BS)(segmap)
    raw_tc = _get_tc_hist(_B - _BS)(
        segmap, jnp.asarray(_ST), jnp.asarray(_U8))
    s = jnp.stack([hist_val_mean, hist_val_std,
                   patch_hist_val_mean, patch_hist_val_std]).reshape(1, 4)
    out = _get_stage2()(raw_sc, raw_tc.reshape(_B - _BS, 16),
                        jnp.asarray(_G5), jnp.asarray(_G20),
                        jnp.asarray(_G5T), jnp.asarray(_G20T),
                        jnp.asarray(_B5), jnp.asarray(_B20),
                        hist_mean.reshape(1, _C), hist_invcov,
                        patch_hist_mean.reshape(1, 4 * _C), patch_hist_invcov, s)
    return out.reshape(_B)


# speedup vs baseline: 1.5546x; 1.5546x over previous
import functools

import numpy as np
import jax
import jax.numpy as jnp
from jax import lax
from jax.experimental import pallas as pl
from jax.experimental.pallas import tpu as pltpu
from jax.experimental.pallas import tpu_sc as plsc

_B = 128
_C = 5
_S = 256
_H = 128
_L = 16
_NC, _NS = 2, 16
_NW = _NC * _NS
_RCH = 32
_CPT = _H // _RCH
_CPW = (_B * 2 * _CPT) // _NW
_HSLOTS = 8 * _L


def _sc_hist_body(cpw, seg_hbm, out_hbm, buf, hist, sem0, sem1):
    wid = lax.axis_index("s") * _NC + lax.axis_index("c")
    g0 = wid * cpw
    one = jnp.ones((_L,), jnp.float32)
    zf = jnp.zeros((_L,), jnp.float32)
    sems = (sem0, sem1)

    def chunk_addr(gg):
        b = gg >> 3
        tb = (gg >> 2) & 1
        r0 = tb * _H + (gg & 3) * _RCH
        return b, tb, r0

    def start(gg, slot):
        b, _, r0 = chunk_addr(gg)
        pltpu.async_copy(
            seg_hbm.at[b, :, pl.ds(r0, _RCH), :], buf.at[slot], sems[slot])

    def wait(gg, slot):
        b, _, r0 = chunk_addr(gg)
        pltpu.make_async_copy(
            seg_hbm.at[b, :, pl.ds(r0, _RCH), :], buf.at[slot],
            sems[slot]).wait()

    def compute(slot, accs):
        def sel4(r, col):
            xs = [buf[slot, c, r, pl.ds(col, _L)] for c in range(_C)]
            m = jnp.maximum(
                jnp.maximum(jnp.maximum(xs[0], xs[1]),
                            jnp.maximum(xs[2], xs[3])), xs[4])
            return [jnp.where(xs[cp + 1] == m, one, zf) for cp in range(4)]

        def half(lr):
            def row(r, a4):
                a4 = list(a4)
                for jp in range(4):
                    s_even = sel4(r, lr * _H + (2 * jp) * _L)
                    s_odd = sel4(r, lr * _H + (2 * jp + 1) * _L)
                    for cp in range(4):
                        a4[cp] = a4[cp] + (s_even[cp] + s_odd[cp])
                return tuple(a4)
            return row

        accl = lax.fori_loop(0, _RCH, half(0), tuple(accs[0:4]))
        accr = lax.fori_loop(0, _RCH, half(1), tuple(accs[4:8]))
        return accl + accr

    start(g0, 0)
    start(g0 + 1, 1)

    def outer(gp, accs):
        for slot in range(2):
            g = gp * 2 + slot
            gg = g0 + g
            wait(gg, slot)
            accs = compute(slot, accs)

            @pl.when(g + 2 < cpw)
            def _():
                start(gg + 2, slot)

            b, tb, _ = chunk_addr(gg)
            flush = (gg & 3) == 3

            @pl.when(flush)
            def _():
                for k in range(8):
                    hist[pl.ds(k * _L, _L)] = accs[k]
                pltpu.sync_copy(hist, out_hbm.at[b, pl.ds(tb * _HSLOTS, _HSLOTS)])

            accs = tuple(jnp.where(flush, zf, a) for a in accs)
        return accs

    lax.fori_loop(0, cpw // 2, outer, tuple(zf for _ in range(8)))


@functools.cache
def _get_sc_hist(bs):
    cpw = (bs * 2 * _CPT) // _NW
    return pl.kernel(
        functools.partial(_sc_hist_body, cpw),
        out_type=jax.ShapeDtypeStruct((bs, 2 * _HSLOTS), jnp.float32),
        mesh=plsc.VectorSubcoreMesh(core_axis_name="c", subcore_axis_name="s"),
        scratch_types=[
            pltpu.VMEM((2, _C, _RCH, _S), jnp.float32),
            pltpu.VMEM((_HSLOTS,), jnp.float32),
            pltpu.SemaphoreType.DMA,
            pltpu.SemaphoreType.DMA,
        ],
        compiler_params=pltpu.CompilerParams(needs_layout_passes=False),
    )


def _build_tc_selectors():
    st = np.zeros((_S, 2), np.float32)
    st[:_H, 0] = 1.0
    st[_H:, 1] = 1.0
    u8 = np.zeros((8, 4 * _S), np.float32)
    for tb in range(2):
        for cp in range(4):
            for r in range(tb * _H, (tb + 1) * _H):
                u8[cp * 2 + tb, cp * _S + r] = 1.0
    return st, u8


_ST, _U8 = _build_tc_selectors()


def _tc_hist_body(seg_ref, st_ref, u8_ref, o_ref):
    x = seg_ref[0]
    m = jnp.max(x, axis=0)
    eq = (x[1:] == m[None]).astype(jnp.float32)
    e = eq.reshape(4 * _S, _S)
    a = jnp.dot(e, st_ref[...], preferred_element_type=jnp.float32)
    o_ref[0] = jnp.dot(u8_ref[...], a, preferred_element_type=jnp.float32)


@functools.cache
def _get_tc_hist(bt):
    b0 = _B - bt
    return pl.pallas_call(
        _tc_hist_body,
        grid=(bt,),
        in_specs=[
            pl.BlockSpec((1, _C, _S, _S), lambda i: (i + b0, 0, 0, 0)),
            pl.BlockSpec((_S, 2), lambda i: (0, 0)),
            pl.BlockSpec((8, 4 * _S), lambda i: (0, 0)),
        ],
        out_specs=pl.BlockSpec((1, 8, 2), lambda i: (i, 0, 0)),
        out_shape=jax.ShapeDtypeStruct((bt, 8, 2), jnp.float32),
    )


def _build_selectors():
    g5 = np.zeros((2 * _HSLOTS, _C), np.float32)
    g20 = np.zeros((2 * _HSLOTS, 4 * _C), np.float32)
    g5t = np.zeros((16, _C), np.float32)
    g20t = np.zeros((16, 4 * _C), np.float32)
    b5 = np.zeros((1, _C), np.float32)
    b20 = np.zeros((1, 4 * _C), np.float32)
    b5[0, 0] = 1.0
    for tb in range(2):
        for lr in range(2):
            q = lr * 2 + tb
            b20[0, q * _C] = 1.0
            for cp in range(_C - 1):
                k = cp * 4 + tb * 2 + lr
                g5t[k, cp + 1] = 1.0 / float(_S * _S)
                g5t[k, 0] = -1.0 / float(_S * _S)
                g20t[k, q * _C + cp + 1] = 1.0 / float(_H * _H)
                g20t[k, q * _C] = -1.0 / float(_H * _H)
                for ln in range(_L):
                    j = tb * _HSLOTS + (lr * 4 + cp) * _L + ln
                    g5[j, cp + 1] = 1.0 / float(_S * _S)
                    g5[j, 0] = -1.0 / float(_S * _S)
                    g20[j, q * _C + cp + 1] = 1.0 / float(_H * _H)
                    g20[j, q * _C] = -1.0 / float(_H * _H)
    return g5, g20, g5t, g20t, b5, b20


_G5, _G20, _G5T, _G20T, _B5, _B20 = _build_selectors()


def _dot(a, b):
    return jnp.dot(a, b, preferred_element_type=jnp.float32)


def _count_dot(y, g):
    hi = jnp.floor(y * (1.0 / 128.0))
    lo = y - hi * 128.0
    return _dot(hi, g) * 128.0 + _dot(lo, g)


def _stage2_body(ysc_ref, ytc_ref, g5_ref, g20_ref, g5t_ref, g20t_ref,
                 b5_ref, b20_ref, m5_ref, iv5_ref, m20_ref, iv20_ref,
                 s_ref, o_ref):
    ysc = ysc_ref[...]
    ytc = ytc_ref[...]
    u5 = jnp.concatenate(
        [_count_dot(ysc, g5_ref[...]), _count_dot(ytc, g5t_ref[...])], axis=0)
    u20 = jnp.concatenate(
        [_count_dot(ysc, g20_ref[...]), _count_dot(ytc, g20t_ref[...])], axis=0)
    d5 = u5 + (b5_ref[...] - m5_ref[...])
    d20 = u20 + (b20_ref[...] - m20_ref[...])
    q5 = jnp.sum(d5 * _dot(d5, iv5_ref[...]), axis=1, keepdims=True)
    q20 = jnp.sum(d20 * _dot(d20, iv20_ref[...]), axis=1, keepdims=True)
    hvm = s_ref[:, 0:1]
    hvs = s_ref[:, 1:2]
    phvm = s_ref[:, 2:3]
    phvs = s_ref[:, 3:4]
    o_ref[...] = (jnp.sqrt(q5) - hvm) / hvs + (jnp.sqrt(q20) - phvm) / phvs


@functools.cache
def _get_stage2():
    return pl.pallas_call(
        _stage2_body,
        out_shape=jax.ShapeDtypeStruct((_B, 1), jnp.float32),
    )


_BS = 72


def kernel(segmap, hist_mean, hist_invcov, patch_hist_mean, patch_hist_invcov,
           hist_val_mean, hist_val_std, patch_hist_val_mean, patch_hist_val_std):
    raw_sc = _get_sc_hist(_BS)(segmap)
    raw_tc = _get_tc_hist(_B - _BS)(
        segmap, jnp.asarray(_ST), jnp.asarray(_U8))
    s = jnp.stack([hist_val_mean, hist_val_std,
                   patch_hist_val_mean, patch_hist_val_std]).reshape(1, 4)
    out = _get_stage2()(raw_sc, raw_tc.reshape(_B - _BS, 16),
                        jnp.asarray(_G5), jnp.asarray(_G20),
                        jnp.asarray(_G5T), jnp.asarray(_G20T),
                        jnp.asarray(_B5), jnp.asarray(_B20),
                        hist_mean.reshape(1, _C), hist_invcov,
                        patch_hist_mean.reshape(1, 4 * _C), patch_hist_invcov, s)
    return out.reshape(_B)


# speedup vs baseline: 1.6596x; 1.0676x over previous
import functools

import numpy as np
import jax
import jax.numpy as jnp
from jax import lax
from jax.experimental import pallas as pl
from jax.experimental.pallas import tpu as pltpu
from jax.experimental.pallas import tpu_sc as plsc

_B = 128
_C = 5
_S = 256
_H = 128
_L = 16
_NC, _NS = 2, 16
_NW = _NC * _NS
_RCH = 32
_CPT = _H // _RCH
_CPW = (_B * 2 * _CPT) // _NW
_HSLOTS = 8 * _L


def _sc_hist_body(cpw, seg_hbm, out_hbm, buf, hist, sem0, sem1):
    wid = lax.axis_index("s") * _NC + lax.axis_index("c")
    g0 = wid * cpw
    one = jnp.ones((_L,), jnp.float32)
    zf = jnp.zeros((_L,), jnp.float32)
    sems = (sem0, sem1)

    def chunk_addr(gg):
        b = gg >> 3
        tb = (gg >> 2) & 1
        r0 = tb * _H + (gg & 3) * _RCH
        return b, tb, r0

    def start(gg, slot):
        b, _, r0 = chunk_addr(gg)
        pltpu.async_copy(
            seg_hbm.at[b, :, pl.ds(r0, _RCH), :], buf.at[slot], sems[slot])

    def wait(gg, slot):
        b, _, r0 = chunk_addr(gg)
        pltpu.make_async_copy(
            seg_hbm.at[b, :, pl.ds(r0, _RCH), :], buf.at[slot],
            sems[slot]).wait()

    def compute(slot, accs):
        def sel4(r, col):
            xs = [buf[slot, c, r, pl.ds(col, _L)] for c in range(_C)]
            m = jnp.maximum(
                jnp.maximum(jnp.maximum(xs[0], xs[1]),
                            jnp.maximum(xs[2], xs[3])), xs[4])
            return [jnp.where(xs[cp + 1] == m, one, zf) for cp in range(4)]

        def half(lr):
            def row(r, a4):
                a4 = list(a4)
                for jp in range(4):
                    s_even = sel4(r, lr * _H + (2 * jp) * _L)
                    s_odd = sel4(r, lr * _H + (2 * jp + 1) * _L)
                    for cp in range(4):
                        a4[cp] = a4[cp] + (s_even[cp] + s_odd[cp])
                return tuple(a4)
            return row

        accl = lax.fori_loop(0, _RCH, half(0), tuple(accs[0:4]))
        accr = lax.fori_loop(0, _RCH, half(1), tuple(accs[4:8]))
        return accl + accr

    start(g0, 0)
    start(g0 + 1, 1)

    def outer(gp, accs):
        for slot in range(2):
            g = gp * 2 + slot
            gg = g0 + g
            wait(gg, slot)
            accs = compute(slot, accs)

            @pl.when(g + 2 < cpw)
            def _():
                start(gg + 2, slot)

            b, tb, _ = chunk_addr(gg)
            flush = (gg & 3) == 3

            @pl.when(flush)
            def _():
                for k in range(8):
                    hist[pl.ds(k * _L, _L)] = accs[k]
                pltpu.sync_copy(hist, out_hbm.at[b, pl.ds(tb * _HSLOTS, _HSLOTS)])

            accs = tuple(jnp.where(flush, zf, a) for a in accs)
        return accs

    lax.fori_loop(0, cpw // 2, outer, tuple(zf for _ in range(8)))


@functools.cache
def _get_sc_hist(bs):
    cpw = (bs * 2 * _CPT) // _NW
    return pl.kernel(
        functools.partial(_sc_hist_body, cpw),
        out_type=jax.ShapeDtypeStruct((bs, 2 * _HSLOTS), jnp.float32),
        mesh=plsc.VectorSubcoreMesh(core_axis_name="c", subcore_axis_name="s"),
        scratch_types=[
            pltpu.VMEM((2, _C, _RCH, _S), jnp.float32),
            pltpu.VMEM((_HSLOTS,), jnp.float32),
            pltpu.SemaphoreType.DMA,
            pltpu.SemaphoreType.DMA,
        ],
        compiler_params=pltpu.CompilerParams(needs_layout_passes=False),
    )


def _build_tc_selectors():
    st = np.zeros((_S, 2), np.float32)
    st[:_H, 0] = 1.0
    st[_H:, 1] = 1.0
    u8 = np.zeros((8, 4 * _S), np.float32)
    for tb in range(2):
        for cp in range(4):
            for r in range(tb * _H, (tb + 1) * _H):
                u8[cp * 2 + tb, cp * _S + r] = 1.0
    return st, u8


_ST, _U8 = _build_tc_selectors()


def _tc_hist_body(seg_ref, st_ref, u8_ref, o_ref):
    x = seg_ref[0]
    m = jnp.max(x, axis=0)
    eq = (x[1:] == m[None]).astype(jnp.float32)
    e = eq.reshape(4 * _S, _S)
    a = jnp.dot(e, st_ref[...], preferred_element_type=jnp.float32)
    o_ref[0] = jnp.dot(u8_ref[...], a, preferred_element_type=jnp.float32)


@functools.cache
def _get_tc_hist(bt):
    b0 = _B - bt
    return pl.pallas_call(
        _tc_hist_body,
        grid=(bt,),
        in_specs=[
            pl.BlockSpec((1, _C, _S, _S), lambda i: (i + b0, 0, 0, 0)),
            pl.BlockSpec((_S, 2), lambda i: (0, 0)),
            pl.BlockSpec((8, 4 * _S), lambda i: (0, 0)),
        ],
        out_specs=pl.BlockSpec((1, 8, 2), lambda i: (i, 0, 0)),
        out_shape=jax.ShapeDtypeStruct((bt, 8, 2), jnp.float32),
    )


def _build_selectors():
    g5 = np.zeros((2 * _HSLOTS, _C), np.float32)
    g20 = np.zeros((2 * _HSLOTS, 4 * _C), np.float32)
    g5t = np.zeros((16, _C), np.float32)
    g20t = np.zeros((16, 4 * _C), np.float32)
    b5 = np.zeros((1, _C), np.float32)
    b20 = np.zeros((1, 4 * _C), np.float32)
    b5[0, 0] = 1.0
    for tb in range(2):
        for lr in range(2):
            q = lr * 2 + tb
            b20[0, q * _C] = 1.0
            for cp in range(_C - 1):
                k = cp * 4 + tb * 2 + lr
                g5t[k, cp + 1] = 1.0 / float(_S * _S)
                g5t[k, 0] = -1.0 / float(_S * _S)
                g20t[k, q * _C + cp + 1] = 1.0 / float(_H * _H)
                g20t[k, q * _C] = -1.0 / float(_H * _H)
                for ln in range(_L):
                    j = tb * _HSLOTS + (lr * 4 + cp) * _L + ln
                    g5[j, cp + 1] = 1.0 / float(_S * _S)
                    g5[j, 0] = -1.0 / float(_S * _S)
                    g20[j, q * _C + cp + 1] = 1.0 / float(_H * _H)
                    g20[j, q * _C] = -1.0 / float(_H * _H)
    return g5, g20, g5t, g20t, b5, b20


_G5, _G20, _G5T, _G20T, _B5, _B20 = _build_selectors()


def _dot(a, b):
    return jnp.dot(a, b, preferred_element_type=jnp.float32)


def _count_dot(y, g):
    hi = jnp.floor(y * (1.0 / 128.0))
    lo = y - hi * 128.0
    return _dot(hi, g) * 128.0 + _dot(lo, g)


def _stage2_body(ysc_ref, ytc_ref, g5_ref, g20_ref, g5t_ref, g20t_ref,
                 b5_ref, b20_ref, m5_ref, iv5_ref, m20_ref, iv20_ref,
                 s_ref, o_ref):
    ysc = ysc_ref[...]
    ytc = ytc_ref[...]
    u5 = jnp.concatenate(
        [_count_dot(ysc, g5_ref[...]), _count_dot(ytc, g5t_ref[...])], axis=0)
    u20 = jnp.concatenate(
        [_count_dot(ysc, g20_ref[...]), _count_dot(ytc, g20t_ref[...])], axis=0)
    d5 = u5 + (b5_ref[...] - m5_ref[...])
    d20 = u20 + (b20_ref[...] - m20_ref[...])
    q5 = jnp.sum(d5 * _dot(d5, iv5_ref[...]), axis=1, keepdims=True)
    q20 = jnp.sum(d20 * _dot(d20, iv20_ref[...]), axis=1, keepdims=True)
    hvm = s_ref[:, 0:1]
    hvs = s_ref[:, 1:2]
    phvm = s_ref[:, 2:3]
    phvs = s_ref[:, 3:4]
    o_ref[...] = (jnp.sqrt(q5) - hvm) / hvs + (jnp.sqrt(q20) - phvm) / phvs


@functools.cache
def _get_stage2():
    return pl.pallas_call(
        _stage2_body,
        out_shape=jax.ShapeDtypeStruct((_B, 1), jnp.float32),
    )


_BS = 80


def kernel(segmap, hist_mean, hist_invcov, patch_hist_mean, patch_hist_invcov,
           hist_val_mean, hist_val_std, patch_hist_val_mean, patch_hist_val_std):
    raw_sc = _get_sc_hist(_BS)(segmap)
    raw_tc = _get_tc_hist(_B - _BS)(
        segmap, jnp.asarray(_ST), jnp.asarray(_U8))
    s = jnp.stack([hist_val_mean, hist_val_std,
                   patch_hist_val_mean, patch_hist_val_std]).reshape(1, 4)
    out = _get_stage2()(raw_sc, raw_tc.reshape(_B - _BS, 16),
                        jnp.asarray(_G5), jnp.asarray(_G20),
                        jnp.asarray(_G5T), jnp.asarray(_G20T),
                        jnp.asarray(_B5), jnp.asarray(_B20),
                        hist_mean.reshape(1, _C), hist_invcov,
                        patch_hist_mean.reshape(1, 4 * _C), patch_hist_invcov, s)
    return out.reshape(_B)
